# Initial kernel scaffold; baseline (speedup 1.0000x reference)
import jax
import jax.numpy as jnp
from jax.experimental import pallas as pl


def kernel(x, edge_index, edge_attr, batch, W_in, b_in, g_in, bt_in, W_e, b_e, g_e, bt_e, W_c, eps, g_n, bt_n, W1, b1, g1, bt1, W2, b2, g2, bt2):
    raise NotImplementedError("write your pallas kernel here")


# speedup vs baseline: 1.0454x; 1.0454x over previous
import functools

import jax
import jax.numpy as jnp
from jax import lax
from jax.experimental import pallas as pl
from jax.experimental.pallas import tpu as pltpu
from jax.experimental.pallas import tpu_sc as plsc

N_NODES = 10000
N_EDGES = 320000
NHID = 128
NEDGE = 16
NLAYER = 3
NGRAPH = 64
NOUT = 128

NC = 2
NS = 16
NW = NC * NS

CHUNK = 128
EDGES_PER_TILE = 10240
NPAD_EDGES = EDGES_PER_TILE * NW
NCHUNK = EDGES_PER_TILE // CHUNK
AGG_ROWS = 10112
STRIPE = AGG_ROWS // NS

_f32 = jnp.float32


def _h0_body(x_ref, w_ref, b_ref, g_ref, bt_ref, o_ref, oT_ref):
    t = jnp.dot(x_ref[...], w_ref[...], preferred_element_type=_f32) + b_ref[...]
    mu = jnp.mean(t, axis=0, keepdims=True)
    var = jnp.mean((t - mu) ** 2, axis=0, keepdims=True)
    h = jnp.maximum(g_ref[...] * (t - mu) / jnp.sqrt(var + 1e-5) + bt_ref[...], 0.0)
    o_ref[...] = h
    oT_ref[...] = jnp.pad(h, ((0, AGG_ROWS - N_NODES), (0, 0))).T


_STATS_BLK = 2560


def _estats_body(a_ref, we_ref, be_ref, ssum_ref, ssq_ref):
    blk = a_ref[...]

    @pl.when(pl.program_id(0) == 0)
    def _():
        ssum_ref[...] = jnp.zeros_like(ssum_ref)
        ssq_ref[...] = jnp.zeros_like(ssq_ref)

    for l in range(NLAYER):
        t = jnp.dot(blk, we_ref[l], preferred_element_type=_f32) + be_ref[l]
        ssum_ref[l] += jnp.sum(t, axis=0, keepdims=True)
        ssq_ref[l] += jnp.sum(t * t, axis=0, keepdims=True)


_E_BLK = 2048


def _edges_body(a_ref, we_ref, be_ref, ge_ref, bte_ref, ssum_ref, ssq_ref,
                e0_ref, e1_ref, e2_ref):
    blk = a_ref[...]
    outs = (e0_ref, e1_ref, e2_ref)
    for l in range(NLAYER):
        mu = ssum_ref[l] * (1.0 / N_EDGES)
        var = ssq_ref[l] * (1.0 / N_EDGES) - mu * mu
        t = jnp.dot(blk, we_ref[l], preferred_element_type=_f32) + be_ref[l]
        outs[l][...] = jnp.maximum(
            ge_ref[l] * (t - mu) / jnp.sqrt(var + 1e-5) + bte_ref[l], 0.0).T


FPT = NHID // NW
IC = 2048
NIC = NPAD_EDGES // IC


def _mp_body(hT_hbm, eT_hbm, src_hbm, dst_hbm, out_hbm,
             h_v, agg_v, e_v, src_v, dst_v):
    cid = lax.axis_index("c")
    sid = lax.axis_index("s")
    wid = sid * NC + cid
    f0 = wid * FPT

    for j in range(FPT):
        pltpu.sync_copy(hT_hbm.at[pl.ds((f0 + j) * AGG_ROWS, AGG_ROWS)],
                        h_v.at[pl.ds(j * AGG_ROWS, AGG_ROWS)])

    def _zero(i, carry):
        agg_v[pl.ds(i * 16, 16)] = jnp.zeros((16,), _f32)
        return carry

    lax.fori_loop(0, FPT * AGG_ROWS // 16, _zero, 0)

    def _chunk(c, carry):
        pltpu.sync_copy(src_hbm.at[pl.ds(c * IC, IC)], src_v)
        pltpu.sync_copy(dst_hbm.at[pl.ds(c * IC, IC)], dst_v)
        for j in range(FPT):
            pltpu.sync_copy(
                eT_hbm.at[pl.ds((f0 + j) * NPAD_EDGES + c * IC, IC)],
                e_v.at[pl.ds(j * IC, IC)])

        def _group(i, gc):
            sl = pl.ds(i * 16, 16)
            s16 = src_v[sl]
            d16 = dst_v[sl]
            for j in range(FPT):
                hv = plsc.load_gather(h_v, [s16 + jnp.int32(j * AGG_ROWS)])
                m = jnp.maximum(hv + e_v[pl.ds(j * IC + i * 16, 16)], 0.0)
                plsc.addupdate_scatter(agg_v, [d16 + jnp.int32(j * AGG_ROWS)], m)
            return gc

        lax.fori_loop(0, IC // 16, _group, 0)
        return carry

    lax.fori_loop(0, NIC, _chunk, 0)

    for j in range(FPT):
        pltpu.sync_copy(agg_v.at[pl.ds(j * AGG_ROWS, AGG_ROWS)],
                        out_hbm.at[pl.ds((f0 + j) * AGG_ROWS, AGG_ROWS)])


@functools.lru_cache(maxsize=1)
def _get_mp_kernel():
    return pl.kernel(
        _mp_body,
        out_type=jax.ShapeDtypeStruct((NHID * AGG_ROWS,), _f32),
        mesh=plsc.VectorSubcoreMesh(core_axis_name="c", subcore_axis_name="s",
                                    num_cores=NC, num_subcores=NS),
        compiler_params=pltpu.CompilerParams(needs_layout_passes=False),
        scratch_types=[
            pltpu.VMEM((FPT * AGG_ROWS,), _f32),
            pltpu.VMEM((FPT * AGG_ROWS,), _f32),
            pltpu.VMEM((FPT * IC,), _f32),
            pltpu.VMEM((IC,), jnp.int32),
            pltpu.VMEM((IC,), jnp.int32),
        ],
    )


def _update_body(h_ref, aT_ref, wc_ref, eps_ref, g_ref, bt_ref, o_ref, oT_ref):
    h = h_ref[...]
    agg = aT_ref[...].T[:N_NODES]
    z = (1.0 + eps_ref[0, 0]) * h + agg
    t = jnp.dot(z, wc_ref[...], preferred_element_type=_f32)
    mu = jnp.mean(t, axis=0, keepdims=True)
    var = jnp.mean((t - mu) ** 2, axis=0, keepdims=True)
    hn = jnp.maximum(g_ref[...] * (t - mu) / jnp.sqrt(var + 1e-5) + bt_ref[...], 0.0) + h
    o_ref[...] = hn
    oT_ref[...] = jnp.pad(hn, ((0, AGG_ROWS - N_NODES), (0, 0))).T


def _pool_body(h_ref, batch_ref, w1_ref, b1_ref, g1_ref, bt1_ref,
               w2_ref, b2_ref, g2_ref, bt2_ref, o_ref):
    h = h_ref[...]
    b = batch_ref[...]
    gids = lax.broadcasted_iota(jnp.int32, (NGRAPH, N_NODES), 0)
    onehot = jnp.where(gids == b, 1.0, 0.0).astype(_f32)
    pooled = jnp.dot(onehot, h, preferred_element_type=_f32,
                     precision=lax.Precision.HIGHEST)

    t = jnp.dot(pooled, w1_ref[...], preferred_element_type=_f32) + b1_ref[...]
    mu = jnp.mean(t, axis=0, keepdims=True)
    var = jnp.mean((t - mu) ** 2, axis=0, keepdims=True)
    o = jnp.maximum(g1_ref[...] * (t - mu) / jnp.sqrt(var + 1e-5) + bt1_ref[...], 0.0)

    t2 = jnp.dot(o, w2_ref[...], preferred_element_type=_f32) + b2_ref[...]
    mu2 = jnp.mean(t2, axis=0, keepdims=True)
    var2 = jnp.mean((t2 - mu2) ** 2, axis=0, keepdims=True)
    o_ref[...] = g2_ref[...] * (t2 - mu2) / jnp.sqrt(var2 + 1e-5) + bt2_ref[...]


def _row(v):
    return v.reshape(1, -1)


def kernel(x, edge_index, edge_attr, batch, W_in, b_in, g_in, bt_in,
           W_e, b_e, g_e, bt_e, W_c, eps, g_n, bt_n,
           W1, b1, g1, bt1, W2, b2, g2, bt2):
    src = jnp.pad(edge_index[0], (0, NPAD_EDGES - N_EDGES))
    dst = jnp.pad(edge_index[1], (0, NPAD_EDGES - N_EDGES),
                  constant_values=N_NODES)
    ea_pad = jnp.pad(edge_attr, ((0, NPAD_EDGES - N_EDGES), (0, 0)))
    batch2d = batch.reshape(1, N_NODES)

    h, hT = pl.pallas_call(
        _h0_body,
        out_shape=[jax.ShapeDtypeStruct((N_NODES, NHID), _f32),
                   jax.ShapeDtypeStruct((NHID, AGG_ROWS), _f32)],
    )(x, W_in, _row(b_in), _row(g_in), _row(bt_in))

    nblk = N_EDGES // _STATS_BLK
    full = lambda *shape: pl.BlockSpec(shape, lambda i: tuple(0 for _ in shape))
    be3 = b_e.reshape(NLAYER, 1, NHID)
    ssum, ssq = pl.pallas_call(
        _estats_body,
        grid=(nblk,),
        in_specs=[pl.BlockSpec((_STATS_BLK, NEDGE), lambda i: (i, 0)),
                  full(NLAYER, NEDGE, NHID),
                  full(NLAYER, 1, NHID)],
        out_specs=[full(NLAYER, 1, NHID), full(NLAYER, 1, NHID)],
        out_shape=[jax.ShapeDtypeStruct((NLAYER, 1, NHID), _f32),
                   jax.ShapeDtypeStruct((NLAYER, 1, NHID), _f32)],
    )(edge_attr, W_e, be3)

    neblk = NPAD_EDGES // _E_BLK
    e_shape = jax.ShapeDtypeStruct((NHID, NPAD_EDGES), _f32)
    E0, E1, E2 = pl.pallas_call(
        _edges_body,
        grid=(neblk,),
        in_specs=[pl.BlockSpec((_E_BLK, NEDGE), lambda i: (i, 0)),
                  full(NLAYER, NEDGE, NHID),
                  full(NLAYER, 1, NHID),
                  full(NLAYER, 1, NHID),
                  full(NLAYER, 1, NHID),
                  full(NLAYER, 1, NHID),
                  full(NLAYER, 1, NHID)],
        out_specs=[pl.BlockSpec((NHID, _E_BLK), lambda i: (0, i))] * NLAYER,
        out_shape=[e_shape] * NLAYER,
    )(ea_pad, W_e, be3, g_e.reshape(NLAYER, 1, NHID),
      bt_e.reshape(NLAYER, 1, NHID), ssum, ssq)

    for l, E in enumerate((E0, E1, E2)):
        aggT = _get_mp_kernel()(hT.reshape(-1), E.reshape(-1), src, dst)
        h, hT = pl.pallas_call(
            _update_body,
            out_shape=[jax.ShapeDtypeStruct((N_NODES, NHID), _f32),
                       jax.ShapeDtypeStruct((NHID, AGG_ROWS), _f32)],
        )(h, aggT.reshape(NHID, AGG_ROWS), W_c[l], eps[l].reshape(1, 1),
          _row(g_n[l]), _row(bt_n[l]))

    out = pl.pallas_call(
        _pool_body,
        out_shape=jax.ShapeDtypeStruct((NGRAPH, NOUT), _f32),
    )(h, batch2d, W1, _row(b1), _row(g1), _row(bt1),
      W2, _row(b2), _row(g2), _row(bt2))
    return out


# speedup vs baseline: 1.3128x; 1.2558x over previous
import functools

import jax
import jax.numpy as jnp
from jax import lax
from jax.experimental import pallas as pl
from jax.experimental.pallas import tpu as pltpu
from jax.experimental.pallas import tpu_sc as plsc

N_NODES = 10000
N_EDGES = 320000
NHID = 128
NEDGE = 16
NLAYER = 3
NGRAPH = 64
NOUT = 128

NC = 2
NS = 16
NW = NC * NS

CHUNK = 128
EDGES_PER_TILE = 10240
NPAD_EDGES = EDGES_PER_TILE * NW
NCHUNK = EDGES_PER_TILE // CHUNK
AGG_ROWS = 10112
STRIPE = AGG_ROWS // NS

_f32 = jnp.float32


def _h0_body(x_ref, w_ref, b_ref, g_ref, bt_ref, o_ref, oT_ref):
    t = jnp.dot(x_ref[...], w_ref[...], preferred_element_type=_f32) + b_ref[...]
    mu = jnp.mean(t, axis=0, keepdims=True)
    var = jnp.mean((t - mu) ** 2, axis=0, keepdims=True)
    h = jnp.maximum(g_ref[...] * (t - mu) / jnp.sqrt(var + 1e-5) + bt_ref[...], 0.0)
    o_ref[...] = h
    oT_ref[...] = jnp.pad(h, ((0, AGG_ROWS - N_NODES), (0, 0))).T.reshape(
        NW, NHID // NW, AGG_ROWS)


_STATS_BLK = 2560


def _estats_body(a_ref, we_ref, be_ref, ssum_ref, ssq_ref):
    blk = a_ref[...]

    @pl.when(pl.program_id(0) == 0)
    def _():
        ssum_ref[...] = jnp.zeros_like(ssum_ref)
        ssq_ref[...] = jnp.zeros_like(ssq_ref)

    for l in range(NLAYER):
        t = jnp.dot(blk, we_ref[l], preferred_element_type=_f32) + be_ref[l]
        ssum_ref[l] += jnp.sum(t, axis=0, keepdims=True)
        ssq_ref[l] += jnp.sum(t * t, axis=0, keepdims=True)


_E_BLK = 2048


def _edges_body(a_ref, we_ref, be_ref, ge_ref, bte_ref, ssum_ref, ssq_ref,
                e0_ref, e1_ref, e2_ref):
    blk = a_ref[...]
    outs = (e0_ref, e1_ref, e2_ref)
    for l in range(NLAYER):
        mu = ssum_ref[l] * (1.0 / N_EDGES)
        var = ssq_ref[l] * (1.0 / N_EDGES) - mu * mu
        t = jnp.dot(blk, we_ref[l], preferred_element_type=_f32) + be_ref[l]
        outs[l][...] = jnp.maximum(
            ge_ref[l] * (t - mu) / jnp.sqrt(var + 1e-5) + bte_ref[l],
            0.0).T.reshape(NW, NHID // NW, _E_BLK)


FPT = NHID // NW
IC = 2048
NIC = NPAD_EDGES // IC


def _mp_body(hT_hbm, eT_hbm, src_hbm, dst_hbm, out_hbm,
             h_v, agg_v, e_v, src_v, dst_v):
    cid = lax.axis_index("c")
    sid = lax.axis_index("s")
    wid = sid * NC + cid

    pltpu.sync_copy(hT_hbm.at[wid], h_v)

    def _zero(i, carry):
        for j in range(FPT):
            agg_v[j, pl.ds(i * 16, 16)] = jnp.zeros((16,), _f32)
        return carry

    lax.fori_loop(0, AGG_ROWS // 16, _zero, 0)

    jconst = [jnp.full((16,), j, jnp.int32) for j in range(FPT)]

    def _chunk(c, carry):
        pltpu.sync_copy(src_hbm.at[pl.ds(c * IC, IC)], src_v)
        pltpu.sync_copy(dst_hbm.at[pl.ds(c * IC, IC)], dst_v)
        pltpu.sync_copy(eT_hbm.at[wid, :, pl.ds(c * IC, IC)], e_v)

        def _group(g, gc):
            for u in range(4):
                i = g * 4 + u
                sl = pl.ds(i * 16, 16)
                s16 = src_v[sl]
                d16 = dst_v[sl]
                for j in range(FPT):
                    hv = plsc.load_gather(h_v, [jconst[j], s16])
                    m = jnp.maximum(hv + e_v[j, sl], 0.0)
                    plsc.addupdate_scatter(agg_v, [jconst[j], d16], m)
            return gc

        lax.fori_loop(0, IC // 64, _group, 0)
        return carry

    lax.fori_loop(0, NIC, _chunk, 0)

    pltpu.sync_copy(agg_v, out_hbm.at[wid])


@functools.lru_cache(maxsize=1)
def _get_mp_kernel():
    return pl.kernel(
        _mp_body,
        out_type=jax.ShapeDtypeStruct((NW, FPT, AGG_ROWS), _f32),
        mesh=plsc.VectorSubcoreMesh(core_axis_name="c", subcore_axis_name="s",
                                    num_cores=NC, num_subcores=NS),
        compiler_params=pltpu.CompilerParams(needs_layout_passes=False),
        scratch_types=[
            pltpu.VMEM((FPT, AGG_ROWS), _f32),
            pltpu.VMEM((FPT, AGG_ROWS), _f32),
            pltpu.VMEM((FPT, IC), _f32),
            pltpu.VMEM((IC,), jnp.int32),
            pltpu.VMEM((IC,), jnp.int32),
        ],
    )


def _update_body(h_ref, aT_ref, wc_ref, eps_ref, g_ref, bt_ref, o_ref, oT_ref):
    h = h_ref[...]
    agg = aT_ref[...].reshape(NHID, AGG_ROWS).T[:N_NODES]
    z = (1.0 + eps_ref[0, 0]) * h + agg
    t = jnp.dot(z, wc_ref[...], preferred_element_type=_f32)
    mu = jnp.mean(t, axis=0, keepdims=True)
    var = jnp.mean((t - mu) ** 2, axis=0, keepdims=True)
    hn = jnp.maximum(g_ref[...] * (t - mu) / jnp.sqrt(var + 1e-5) + bt_ref[...], 0.0) + h
    o_ref[...] = hn
    oT_ref[...] = jnp.pad(hn, ((0, AGG_ROWS - N_NODES), (0, 0))).T.reshape(
        NW, NHID // NW, AGG_ROWS)


def _pool_body(h_ref, batch_ref, w1_ref, b1_ref, g1_ref, bt1_ref,
               w2_ref, b2_ref, g2_ref, bt2_ref, o_ref):
    h = h_ref[...]
    b = batch_ref[...]
    gids = lax.broadcasted_iota(jnp.int32, (NGRAPH, N_NODES), 0)
    onehot = jnp.where(gids == b, 1.0, 0.0).astype(_f32)
    pooled = jnp.dot(onehot, h, preferred_element_type=_f32,
                     precision=lax.Precision.HIGHEST)

    t = jnp.dot(pooled, w1_ref[...], preferred_element_type=_f32) + b1_ref[...]
    mu = jnp.mean(t, axis=0, keepdims=True)
    var = jnp.mean((t - mu) ** 2, axis=0, keepdims=True)
    o = jnp.maximum(g1_ref[...] * (t - mu) / jnp.sqrt(var + 1e-5) + bt1_ref[...], 0.0)

    t2 = jnp.dot(o, w2_ref[...], preferred_element_type=_f32) + b2_ref[...]
    mu2 = jnp.mean(t2, axis=0, keepdims=True)
    var2 = jnp.mean((t2 - mu2) ** 2, axis=0, keepdims=True)
    o_ref[...] = g2_ref[...] * (t2 - mu2) / jnp.sqrt(var2 + 1e-5) + bt2_ref[...]


def _row(v):
    return v.reshape(1, -1)


def kernel(x, edge_index, edge_attr, batch, W_in, b_in, g_in, bt_in,
           W_e, b_e, g_e, bt_e, W_c, eps, g_n, bt_n,
           W1, b1, g1, bt1, W2, b2, g2, bt2):
    src = jnp.pad(edge_index[0], (0, NPAD_EDGES - N_EDGES))
    dst = jnp.pad(edge_index[1], (0, NPAD_EDGES - N_EDGES),
                  constant_values=N_NODES)
    ea_pad = jnp.pad(edge_attr, ((0, NPAD_EDGES - N_EDGES), (0, 0)))
    batch2d = batch.reshape(1, N_NODES)

    h, hT = pl.pallas_call(
        _h0_body,
        out_shape=[jax.ShapeDtypeStruct((N_NODES, NHID), _f32),
                   jax.ShapeDtypeStruct((NW, FPT, AGG_ROWS), _f32)],
    )(x, W_in, _row(b_in), _row(g_in), _row(bt_in))

    nblk = N_EDGES // _STATS_BLK
    full = lambda *shape: pl.BlockSpec(shape, lambda i: tuple(0 for _ in shape))
    be3 = b_e.reshape(NLAYER, 1, NHID)
    ssum, ssq = pl.pallas_call(
        _estats_body,
        grid=(nblk,),
        in_specs=[pl.BlockSpec((_STATS_BLK, NEDGE), lambda i: (i, 0)),
                  full(NLAYER, NEDGE, NHID),
                  full(NLAYER, 1, NHID)],
        out_specs=[full(NLAYER, 1, NHID), full(NLAYER, 1, NHID)],
        out_shape=[jax.ShapeDtypeStruct((NLAYER, 1, NHID), _f32),
                   jax.ShapeDtypeStruct((NLAYER, 1, NHID), _f32)],
    )(edge_attr, W_e, be3)

    neblk = NPAD_EDGES // _E_BLK
    e_shape = jax.ShapeDtypeStruct((NW, FPT, NPAD_EDGES), _f32)
    E0, E1, E2 = pl.pallas_call(
        _edges_body,
        grid=(neblk,),
        in_specs=[pl.BlockSpec((_E_BLK, NEDGE), lambda i: (i, 0)),
                  full(NLAYER, NEDGE, NHID),
                  full(NLAYER, 1, NHID),
                  full(NLAYER, 1, NHID),
                  full(NLAYER, 1, NHID),
                  full(NLAYER, 1, NHID),
                  full(NLAYER, 1, NHID)],
        out_specs=[pl.BlockSpec((NW, FPT, _E_BLK), lambda i: (0, 0, i))] * NLAYER,
        out_shape=[e_shape] * NLAYER,
    )(ea_pad, W_e, be3, g_e.reshape(NLAYER, 1, NHID),
      bt_e.reshape(NLAYER, 1, NHID), ssum, ssq)

    for l, E in enumerate((E0, E1, E2)):
        aggT = _get_mp_kernel()(hT, E, src, dst)
        h, hT = pl.pallas_call(
            _update_body,
            out_shape=[jax.ShapeDtypeStruct((N_NODES, NHID), _f32),
                       jax.ShapeDtypeStruct((NW, FPT, AGG_ROWS), _f32)],
        )(h, aggT, W_c[l], eps[l].reshape(1, 1),
          _row(g_n[l]), _row(bt_n[l]))

    out = pl.pallas_call(
        _pool_body,
        out_shape=jax.ShapeDtypeStruct((NGRAPH, NOUT), _f32),
    )(h, batch2d, W1, _row(b1), _row(g1), _row(bt1),
      W2, _row(b2), _row(g2), _row(bt2))
    return out
